# baseline (speedup 1.0000x reference)
import dataclasses
import functools

import jax
import jax.numpy as jnp
from jax import lax
from jax.experimental import pallas as pl
from jax.experimental.pallas import tpu as pltpu
from jax.experimental.pallas import tpu_sc as plsc

NC = 2
NS = 16
L = 16
GL = 128
ET = 10240


def _aggregate(rows, cols, v, unique_nodes_list, W):
    E = v.shape[0]
    V, D = W.shape
    DH = D // NC
    B = 10000
    ET_LAST = E - (NS - 1) * ET
    CR = 200

    Wf = W.reshape(2 * V, DH)
    zeros = jnp.zeros((1000, DH), jnp.float32)

    mesh = plsc.VectorSubcoreMesh(core_axis_name="c", subcore_axis_name="s")

    cp = pltpu.CompilerParams()
    if "needs_layout_passes" in pltpu.CompilerParams.__dataclass_fields__:
        cp = dataclasses.replace(cp, needs_layout_passes=False)

    @functools.partial(
        pl.kernel,
        out_type=jax.ShapeDtypeStruct((B, D), jnp.float32),
        mesh=mesh,
        compiler_params=cp,
        scratch_types=[
            pltpu.VMEM((ET,), jnp.int32),
            [pltpu.VMEM((GL,), jnp.int32) for _ in range(2)],
            [pltpu.VMEM((GL,), jnp.int32) for _ in range(2)],
            [pltpu.VMEM((GL,), jnp.int32) for _ in range(2)],
            [pltpu.VMEM((GL,), jnp.float32) for _ in range(2)],
            [pltpu.VMEM((GL, DH), jnp.float32) for _ in range(2)],
            pltpu.VMEM_SHARED((B, DH), jnp.float32),
            [pltpu.SemaphoreType.DMA for _ in range(2)],
            [pltpu.SemaphoreType.DMA for _ in range(2)],
            [pltpu.SemaphoreType.DMA for _ in range(2)],
            [pltpu.SemaphoreType.DMA for _ in range(2)],
            [pltpu.SemaphoreType.DMA for _ in range(2)],
        ],
    )
    def run(rows_hbm, cols_hbm, v_hbm, unl_hbm, wf_hbm, z_hbm, out_hbm,
            ci_all, bn, ids, rib, vvb, gbuf, acc,
            gsem, ssem, esem, psem, vsem):
        core = lax.axis_index("c")
        sub = lax.axis_index("s")
        e0 = sub * ET
        is_last = sub == NS - 1
        ng = jnp.where(is_last, ET_LAST // GL, ET // GL)

        @pl.when(sub < B // 1000)
        def _zinit():
            pltpu.sync_copy(z_hbm, acc.at[pl.ds(sub * 1000, 1000)])

        @pl.when(jnp.logical_not(is_last))
        def _ldmain():
            pltpu.sync_copy(cols_hbm.at[pl.ds(e0, ET)], ci_all)

        @pl.when(is_last)
        def _ldtail():
            pltpu.sync_copy(cols_hbm.at[pl.ds(e0, ET_LAST)],
                            ci_all.at[pl.ds(0, ET_LAST)])

        plsc.subcore_barrier()

        def eg(g, p):
            pltpu.async_copy(
                unl_hbm.at[ci_all.at[pl.ds(g * GL, GL)]], bn[p], esem[p])

        def eg_wait(p):
            pltpu.make_async_copy(unl_hbm.at[ci_all.at[pl.ds(0, GL)]],
                                  bn[p], esem[p]).wait()

        def transform(p):
            for s in range(GL // L):
                ids[p][pl.ds(s * L, L)] = bn[p][pl.ds(s * L, L)] * 2 + core

        def pf(g, p):
            pltpu.async_copy(rows_hbm.at[pl.ds(e0 + g * GL, GL)],
                             rib[p], psem[p])
            pltpu.async_copy(v_hbm.at[pl.ds(e0 + g * GL, GL)],
                             vvb[p], vsem[p])

        def pf_wait(p):
            pltpu.make_async_copy(rows_hbm.at[pl.ds(0, GL)],
                                  rib[p], psem[p]).wait()
            pltpu.make_async_copy(v_hbm.at[pl.ds(0, GL)],
                                  vvb[p], vsem[p]).wait()

        def gather(p):
            pltpu.async_copy(wf_hbm.at[ids[p]], gbuf[p], gsem[p])

        def gather_wait(p):
            pltpu.make_async_copy(wf_hbm.at[ids[p]], gbuf[p], gsem[p]).wait()

        def weight(p):
            for g in range(0, GL, L):
                vvec = vvb[p][pl.ds(g, L)]
                for lane in range(L):
                    s = vvec[lane]
                    r = g + lane
                    for j in range(0, DH, L):
                        gbuf[p][r, pl.ds(j, L)] = gbuf[p][r, pl.ds(j, L)] * s

        def scat(p):
            pltpu.async_copy(gbuf[p], acc.at[rib[p]], ssem[p], add=True)

        def scat_wait(p):
            pltpu.make_async_copy(gbuf[p], acc.at[rib[p]], ssem[p]).wait()

        pf(0, 0)
        pf(1, 1)
        eg(0, 0)
        eg(1, 1)
        eg_wait(0)
        transform(0)
        eg(2, 0)
        eg_wait(1)
        transform(1)
        eg(3, 1)
        gather(0)
        gather(1)

        @pl.loop(0, (ET // GL) // 2, step=1)
        def _body(i):
            a = 2 * i + 2

            @pl.when(a <= ng)
            def _active():
                for p in range(2):
                    g_next = a + p
                    pf_wait(p)
                    gather_wait(p)
                    weight(p)
                    scat(p)

                    @pl.when(g_next < ng)
                    def _eg_consume():
                        eg_wait(p)
                        transform(p)

                    @pl.when(g_next + 2 < ng)
                    def _eg_issue():
                        eg(g_next + 2, p)

                    @pl.when(g_next < ng)
                    def _prep():
                        scat_wait(p)
                        pf(g_next, p)
                        gather(p)

                    @pl.when(g_next >= ng)
                    def _drain():
                        scat_wait(p)

        plsc.subcore_barrier()

        @pl.when(sub < B // 1000)
        def _copy_out():
            @pl.loop(0, 1000, step=CR)
            def _out(k):
                @pl.when(core == 0)
                def _o0():
                    pltpu.sync_copy(
                        acc.at[pl.ds(sub * 1000 + k, CR)],
                        out_hbm.at[pl.ds(sub * 1000 + k, CR), pl.ds(0, DH)])

                @pl.when(core == 1)
                def _o1():
                    pltpu.sync_copy(
                        acc.at[pl.ds(sub * 1000 + k, CR)],
                        out_hbm.at[pl.ds(sub * 1000 + k, CR), pl.ds(DH, DH)])

    return run(rows, cols, v, unique_nodes_list, Wf, zeros)


def kernel(nodes_real, indices, v, unique_nodes_list, num_sample, W):
    del num_sample
    assert nodes_real.shape[0] == 10000
    rows = indices[0].astype(jnp.int32)
    cols = indices[1].astype(jnp.int32)
    return _aggregate(rows, cols, v, unique_nodes_list.astype(jnp.int32), W)


# speedup vs baseline: 1.0847x; 1.0847x over previous
import dataclasses
import functools

import jax
import jax.numpy as jnp
from jax import lax
from jax.experimental import pallas as pl
from jax.experimental.pallas import tpu as pltpu
from jax.experimental.pallas import tpu_sc as plsc

NC = 2
NS = 16
L = 16
CH = 32
NB = 4
PB = 64
ET = 10240


def _aggregate(rows, cols, v, unique_nodes_list, W):
    E = v.shape[0]
    V, D = W.shape
    DH = D // NC
    B = 10000
    ET_LAST = E - (NS - 1) * ET
    CR = 200

    Wf = W.reshape(2 * V, DH)
    zeros = jnp.zeros((1000, DH), jnp.float32)

    mesh = plsc.VectorSubcoreMesh(core_axis_name="c", subcore_axis_name="s")

    cp = pltpu.CompilerParams()
    if "needs_layout_passes" in pltpu.CompilerParams.__dataclass_fields__:
        cp = dataclasses.replace(cp, needs_layout_passes=False)

    @functools.partial(
        pl.kernel,
        out_type=jax.ShapeDtypeStruct((B, D), jnp.float32),
        mesh=mesh,
        compiler_params=cp,
        scratch_types=[
            pltpu.VMEM((ET,), jnp.int32),
            pltpu.VMEM((ET,), jnp.int32),
            pltpu.VMEM((ET,), jnp.float32),
            [pltpu.VMEM((PB,), jnp.int32) for _ in range(NB)],
            [pltpu.VMEM((CH,), jnp.int32) for _ in range(NB)],
            [pltpu.VMEM((CH, DH), jnp.float32) for _ in range(NB)],
            pltpu.VMEM_SHARED((B, DH), jnp.float32),
            [pltpu.SemaphoreType.DMA for _ in range(NB)],
            [pltpu.SemaphoreType.DMA for _ in range(NB)],
        ],
    )
    def run(rows_hbm, cols_hbm, v_hbm, unl_hbm, wf_hbm, z_hbm, out_hbm,
            ci_all, ri_all, vv_all, bn, ir, gbuf, acc, gsem, ssem):
        core = lax.axis_index("c")
        sub = lax.axis_index("s")
        e0 = sub * ET
        is_last = sub == NS - 1
        nchunk = jnp.where(is_last, ET_LAST // CH, ET // CH)
        nblk = jnp.where(is_last, ET_LAST // PB, ET // PB)

        @pl.when(sub < B // 1000)
        def _zinit():
            pltpu.async_copy(z_hbm, acc.at[pl.ds(sub * 1000, 1000)], ssem[3])

        @pl.when(jnp.logical_not(is_last))
        def _ldmain():
            pltpu.async_copy(rows_hbm.at[pl.ds(e0, ET)], ri_all, ssem[0])
            pltpu.async_copy(v_hbm.at[pl.ds(e0, ET)], vv_all, ssem[1])
            pltpu.sync_copy(cols_hbm.at[pl.ds(e0, ET)], ci_all)

        @pl.when(is_last)
        def _ldtail():
            pltpu.async_copy(rows_hbm.at[pl.ds(e0, ET_LAST)],
                             ri_all.at[pl.ds(0, ET_LAST)], ssem[0])
            pltpu.async_copy(v_hbm.at[pl.ds(e0, ET_LAST)],
                             vv_all.at[pl.ds(0, ET_LAST)], ssem[1])
            pltpu.sync_copy(cols_hbm.at[pl.ds(e0, ET_LAST)],
                            ci_all.at[pl.ds(0, ET_LAST)])

        def eg(kb, q):
            pltpu.async_copy(
                unl_hbm.at[ci_all.at[pl.ds(kb * PB, PB)]], bn[q], gsem[q])

        def eg_wait(q):
            pltpu.make_async_copy(unl_hbm.at[ci_all.at[pl.ds(0, PB)]],
                                  bn[q], gsem[q]).wait()

        def wb(kb, q):
            for s in range(PB // L):
                ci_all[pl.ds(kb * PB + s * L, L)] = (
                    bn[q][pl.ds(s * L, L)] * 2 + core)

        for q in range(NB):
            eg(q, q)

        @pl.loop(NB, nblk, step=NB)
        def _pre(kb):
            for q in range(NB):
                eg_wait(q)
                wb(kb - NB + q, q)
                eg(kb + q, q)

        for q in range(NB):
            eg_wait(q)
            wb(nblk - NB + q, q)

        @pl.when(sub < B // 1000)
        def _zdrain():
            pltpu.make_async_copy(
                z_hbm, acc.at[pl.ds(sub * 1000, 1000)], ssem[3]).wait()

        @pl.when(jnp.logical_not(is_last))
        def _lddrain():
            pltpu.make_async_copy(rows_hbm.at[pl.ds(e0, ET)],
                                  ri_all, ssem[0]).wait()
            pltpu.make_async_copy(v_hbm.at[pl.ds(e0, ET)],
                                  vv_all, ssem[1]).wait()

        @pl.when(is_last)
        def _lddrain2():
            pltpu.make_async_copy(rows_hbm.at[pl.ds(e0, ET_LAST)],
                                  ri_all.at[pl.ds(0, ET_LAST)],
                                  ssem[0]).wait()
            pltpu.make_async_copy(v_hbm.at[pl.ds(e0, ET_LAST)],
                                  vv_all.at[pl.ds(0, ET_LAST)],
                                  ssem[1]).wait()

        plsc.subcore_barrier()

        def compose(c, q):
            for g in range(0, CH, L):
                ir[q][pl.ds(g, L)] = ri_all[pl.ds(c * CH + g, L)]

        def gather(c, q):
            pltpu.async_copy(
                wf_hbm.at[ci_all.at[pl.ds(c * CH, CH)]], gbuf[q], gsem[q])

        def gather_wait(q):
            pltpu.make_async_copy(
                wf_hbm.at[ci_all.at[pl.ds(0, CH)]], gbuf[q], gsem[q]).wait()

        def weight(c, q):
            for g in range(0, CH, L):
                vvec = vv_all[pl.ds(c * CH + g, L)]
                for lane in range(L):
                    s = vvec[lane]
                    r = g + lane
                    for j in range(0, DH, L):
                        gbuf[q][r, pl.ds(j, L)] = gbuf[q][r, pl.ds(j, L)] * s

        def scat(q):
            pltpu.async_copy(gbuf[q], acc.at[ir[q]], ssem[q], add=True)

        def scat_wait(q):
            pltpu.make_async_copy(gbuf[q], acc.at[ir[q]], ssem[q]).wait()

        for q in range(NB):
            compose(q, q)
            gather(q, q)

        @pl.loop(NB, nchunk, step=NB)
        def _body(c):
            for q in range(NB):
                gather_wait(q)
                weight(c - NB + q, q)
                scat(q)

            for q in range(NB):
                scat_wait(q)
                compose(c + q, q)
                gather(c + q, q)

        for q in range(NB):
            gather_wait(q)
            weight(nchunk - NB + q, q)
            scat(q)
        for q in range(NB):
            scat_wait(q)

        plsc.subcore_barrier()

        @pl.when(sub < B // 1000)
        def _copy_out():
            @pl.loop(0, 1000, step=CR)
            def _out(k):
                @pl.when(core == 0)
                def _o0():
                    pltpu.sync_copy(
                        acc.at[pl.ds(sub * 1000 + k, CR)],
                        out_hbm.at[pl.ds(sub * 1000 + k, CR), pl.ds(0, DH)])

                @pl.when(core == 1)
                def _o1():
                    pltpu.sync_copy(
                        acc.at[pl.ds(sub * 1000 + k, CR)],
                        out_hbm.at[pl.ds(sub * 1000 + k, CR), pl.ds(DH, DH)])

    return run(rows, cols, v, unique_nodes_list, Wf, zeros)


def kernel(nodes_real, indices, v, unique_nodes_list, num_sample, W):
    del num_sample
    assert nodes_real.shape[0] == 10000
    rows = indices[0].astype(jnp.int32)
    cols = indices[1].astype(jnp.int32)
    return _aggregate(rows, cols, v, unique_nodes_list.astype(jnp.int32), W)


# speedup vs baseline: 1.1772x; 1.0853x over previous
import dataclasses
import functools

import jax
import jax.numpy as jnp
from jax import lax
from jax.experimental import pallas as pl
from jax.experimental.pallas import tpu as pltpu
from jax.experimental.pallas import tpu_sc as plsc

NC = 2
NS = 16
L = 16
CH = 32
NB = 4
PB = 64
ET = 10240


def _aggregate(rows, cols, v, unique_nodes_list, W):
    E = v.shape[0]
    V, D = W.shape
    DH = D // NC
    B = 10000
    ET_LAST = E - (NS - 1) * ET
    CR = 200

    Wf = W.reshape(2 * V, DH)
    zeros = jnp.zeros((1000, DH), jnp.float32)

    mesh = plsc.VectorSubcoreMesh(core_axis_name="c", subcore_axis_name="s")

    cp = pltpu.CompilerParams()
    if "needs_layout_passes" in pltpu.CompilerParams.__dataclass_fields__:
        cp = dataclasses.replace(cp, needs_layout_passes=False)

    @functools.partial(
        pl.kernel,
        out_type=jax.ShapeDtypeStruct((B, D), jnp.float32),
        mesh=mesh,
        compiler_params=cp,
        scratch_types=[
            pltpu.VMEM((ET,), jnp.int32),
            pltpu.VMEM((ET,), jnp.int32),
            pltpu.VMEM((ET,), jnp.float32),
            [pltpu.VMEM((CH,), jnp.int32) for _ in range(NB)],
            [pltpu.VMEM((CH,), jnp.int32) for _ in range(NB)],
            [pltpu.VMEM((CH,), jnp.int32) for _ in range(NB)],
            [pltpu.VMEM((CH, DH), jnp.float32) for _ in range(NB)],
            pltpu.VMEM_SHARED((B, DH), jnp.float32),
            [pltpu.SemaphoreType.DMA for _ in range(NB)],
            [pltpu.SemaphoreType.DMA for _ in range(NB)],
            [pltpu.SemaphoreType.DMA for _ in range(NB)],
        ],
    )
    def run(rows_hbm, cols_hbm, v_hbm, unl_hbm, wf_hbm, z_hbm, out_hbm,
            ci_all, ri_all, vv_all, bn, idw, ir, gbuf, acc,
            gsem, ssem, esem):
        core = lax.axis_index("c")
        sub = lax.axis_index("s")
        e0 = sub * ET
        is_last = sub == NS - 1
        nchunk = jnp.where(is_last, ET_LAST // CH, ET // CH)
        nblk = jnp.where(is_last, ET_LAST // PB, ET // PB)

        @pl.when(sub < B // 1000)
        def _zinit():
            pltpu.async_copy(z_hbm, acc.at[pl.ds(sub * 1000, 1000)], ssem[3])

        @pl.when(jnp.logical_not(is_last))
        def _ldmain():
            pltpu.async_copy(rows_hbm.at[pl.ds(e0, ET)], ri_all, ssem[0])
            pltpu.async_copy(v_hbm.at[pl.ds(e0, ET)], vv_all, ssem[1])
            pltpu.sync_copy(cols_hbm.at[pl.ds(e0, ET)], ci_all)

        @pl.when(is_last)
        def _ldtail():
            pltpu.async_copy(rows_hbm.at[pl.ds(e0, ET_LAST)],
                             ri_all.at[pl.ds(0, ET_LAST)], ssem[0])
            pltpu.async_copy(v_hbm.at[pl.ds(e0, ET_LAST)],
                             vv_all.at[pl.ds(0, ET_LAST)], ssem[1])
            pltpu.sync_copy(cols_hbm.at[pl.ds(e0, ET_LAST)],
                            ci_all.at[pl.ds(0, ET_LAST)])

        @pl.when(sub < B // 1000)
        def _zdrain():
            pltpu.make_async_copy(
                z_hbm, acc.at[pl.ds(sub * 1000, 1000)], ssem[3]).wait()

        @pl.when(jnp.logical_not(is_last))
        def _lddrain():
            pltpu.make_async_copy(rows_hbm.at[pl.ds(e0, ET)],
                                  ri_all, ssem[0]).wait()
            pltpu.make_async_copy(v_hbm.at[pl.ds(e0, ET)],
                                  vv_all, ssem[1]).wait()

        @pl.when(is_last)
        def _lddrain2():
            pltpu.make_async_copy(rows_hbm.at[pl.ds(e0, ET_LAST)],
                                  ri_all.at[pl.ds(0, ET_LAST)],
                                  ssem[0]).wait()
            pltpu.make_async_copy(v_hbm.at[pl.ds(e0, ET_LAST)],
                                  vv_all.at[pl.ds(0, ET_LAST)],
                                  ssem[1]).wait()

        plsc.subcore_barrier()

        def eg(c, q):
            pltpu.async_copy(
                unl_hbm.at[ci_all.at[pl.ds(c * CH, CH)]], bn[q], esem[q])

        def compose(c, q):
            pltpu.make_async_copy(unl_hbm.at[ci_all.at[pl.ds(0, CH)]],
                                  bn[q], esem[q]).wait()
            for g in range(0, CH, L):
                idw[q][pl.ds(g, L)] = bn[q][pl.ds(g, L)] * 2 + core
                ir[q][pl.ds(g, L)] = ri_all[pl.ds(c * CH + g, L)]

            @pl.when(c + NB < nchunk)
            def _egn():
                eg(c + NB, q)

        def gather(c, q):
            del c
            pltpu.async_copy(wf_hbm.at[idw[q]], gbuf[q], gsem[q])

        def gather_wait(q):
            pltpu.make_async_copy(
                wf_hbm.at[idw[q]], gbuf[q], gsem[q]).wait()

        def weight(c, q):
            for g in range(0, CH, L):
                vvec = vv_all[pl.ds(c * CH + g, L)]
                for lane in range(L):
                    s = vvec[lane]
                    r = g + lane
                    for j in range(0, DH, L):
                        gbuf[q][r, pl.ds(j, L)] = gbuf[q][r, pl.ds(j, L)] * s

        def scat(q):
            pltpu.async_copy(gbuf[q], acc.at[ir[q]], ssem[q], add=True)

        def scat_wait(q):
            pltpu.make_async_copy(gbuf[q], acc.at[ir[q]], ssem[q]).wait()

        for q in range(NB):
            eg(q, q)
        for q in range(NB):
            compose(q, q)
            gather(q, q)

        @pl.loop(NB, nchunk, step=NB)
        def _body(c):
            for q in range(NB):
                gather_wait(q)
                weight(c - NB + q, q)
                scat(q)

            for q in range(NB):
                scat_wait(q)
                compose(c + q, q)
                gather(c + q, q)

        for q in range(NB):
            gather_wait(q)
            weight(nchunk - NB + q, q)
            scat(q)
        for q in range(NB):
            scat_wait(q)

        plsc.subcore_barrier()

        @pl.when(sub < B // 1000)
        def _copy_out():
            @pl.loop(0, 1000, step=CR)
            def _out(k):
                @pl.when(core == 0)
                def _o0():
                    pltpu.sync_copy(
                        acc.at[pl.ds(sub * 1000 + k, CR)],
                        out_hbm.at[pl.ds(sub * 1000 + k, CR), pl.ds(0, DH)])

                @pl.when(core == 1)
                def _o1():
                    pltpu.sync_copy(
                        acc.at[pl.ds(sub * 1000 + k, CR)],
                        out_hbm.at[pl.ds(sub * 1000 + k, CR), pl.ds(DH, DH)])

    return run(rows, cols, v, unique_nodes_list, Wf, zeros)


def kernel(nodes_real, indices, v, unique_nodes_list, num_sample, W):
    del num_sample
    assert nodes_real.shape[0] == 10000
    rows = indices[0].astype(jnp.int32)
    cols = indices[1].astype(jnp.int32)
    return _aggregate(rows, cols, v, unique_nodes_list.astype(jnp.int32), W)
